# baseline (speedup 1.0000x reference)
import functools

import jax
import jax.numpy as jnp
from jax import lax
from jax.experimental import pallas as pl
from jax.experimental.pallas import tpu as pltpu
from jax.experimental.pallas import tpu_sc as plsc

N = 10000
NP = 10240
E = 320000
EP = 322560
HID = 128
H = 8
C = 16
EDGE_DIM = 4
LAYERS = 6
OUT_DIM = 3

RB = 1024
CH = 96
NCHUNK = EP // CH
NWORK = 32
TPW = NCHUNK // NWORK
ROWS_PER_SUB = NP // 16

f32 = jnp.float32


def _ln(t, g, b):
    mu = jnp.mean(t, axis=-1, keepdims=True)
    var = jnp.mean((t - mu) ** 2, axis=-1, keepdims=True)
    return (t - mu) / jnp.sqrt(var + 1e-5) * g + b


def _full(shape):
    return pl.BlockSpec(shape, lambda i: tuple(0 for _ in shape))


def _rows(w):
    return pl.BlockSpec((RB, w), lambda i: (i, 0))


def _pre_part(h, w_ref, acat_ref, xh_ref, tsrc_ref, tdst_ref):
    xh = jnp.dot(h, w_ref[...], preferred_element_type=f32)
    al = jnp.dot(xh, acat_ref[...], preferred_element_type=f32)
    xh_ref[...] = xh
    tsrc_ref[...] = al[:, :16]
    tdst_ref[...] = al[:, 16:]


def _prologue_body(x_ref, w_ref, b_ref, g_ref, bb_ref, wg_ref, acat_ref,
                   o_ref, xh_ref, tsrc_ref, tdst_ref):
    h = jax.nn.relu(jnp.dot(x_ref[...], w_ref[...],
                            preferred_element_type=f32) + b_ref[...])
    h = _ln(h, g_ref[...], bb_ref[...])
    o_ref[...] = h
    _pre_part(h, wg_ref, acat_ref, xh_ref, tsrc_ref, tdst_ref)


def _prologue(x, w, b, g, bb, wg, acat):
    return pl.pallas_call(
        _prologue_body,
        grid=(NP // RB,),
        in_specs=[
            _rows(HID), _full((HID, HID)), _full((1, HID)), _full((1, HID)),
            _full((1, HID)), _full((HID, HID)), _full((HID, 32)),
        ],
        out_specs=[_rows(HID), _rows(HID), _rows(16), _rows(16)],
        out_shape=[
            jax.ShapeDtypeStruct((NP, HID), f32),
            jax.ShapeDtypeStruct((NP, HID), f32),
            jax.ShapeDtypeStruct((NP, 16), f32),
            jax.ShapeDtypeStruct((NP, 16), f32),
        ],
    )(x, w, b, g, bb, wg, acat)


def _ae_body(ea_ref, *refs):
    k_refs = refs[:LAYERS]
    o_refs = refs[LAYERS:2 * LAYERS]
    m_refs = refs[2 * LAYERS:]
    ea = ea_ref[...]
    for l in range(LAYERS):
        r = jnp.dot(ea, k_refs[l][...], preferred_element_type=f32)
        o_refs[l][...] = r
        m_refs[l][...] = jnp.max(r, axis=0).reshape(1, 1, HID)


def _ae_kernel(ea_r, klist):
    EB8 = EP // 8 // 24
    return pl.pallas_call(
        _ae_body,
        grid=(24,),
        in_specs=[pl.BlockSpec((EB8, 32), lambda i: (i, 0))] +
                 [_full((32, HID)) for _ in range(LAYERS)],
        out_specs=[pl.BlockSpec((EB8, HID), lambda i: (i, 0))
                   for _ in range(LAYERS)] +
                  [pl.BlockSpec((1, 1, HID), lambda i: (i, 0, 0))
                   for _ in range(LAYERS)],
        out_shape=[jax.ShapeDtypeStruct((EP // 8, HID), f32)
                   for _ in range(LAYERS)] +
                  [jax.ShapeDtypeStruct((24, 1, HID), f32)
                   for _ in range(LAYERS)],
    )(ea_r, *klist)


def _combine(h_ref, a0_ref, a1_ref, d0_ref, d1_ref, eexp_ref, bg_ref,
             lng_ref, lnb_ref):
    acc = a0_ref[...] + a1_ref[...]
    den = d0_ref[...] + d1_ref[...]
    denx = jnp.dot(den, eexp_ref[...], preferred_element_type=f32)
    g = acc / (denx + 1e-16) + bg_ref[...]
    return _ln(h_ref[...] + g, lng_ref[...], lnb_ref[...])


_COMBINE_SPECS = [
    _rows(HID), _rows(HID), _rows(HID), _rows(16), _rows(16),
    _full((16, HID)), _full((1, HID)), _full((1, HID)), _full((1, HID)),
]


def _postpre_body(h_ref, a0_ref, a1_ref, d0_ref, d1_ref, eexp_ref, bg_ref,
                  lng_ref, lnb_ref, wg_ref, acat_ref,
                  o_ref, xh_ref, tsrc_ref, tdst_ref):
    h = _combine(h_ref, a0_ref, a1_ref, d0_ref, d1_ref, eexp_ref, bg_ref,
                 lng_ref, lnb_ref)
    o_ref[...] = h
    _pre_part(h, wg_ref, acat_ref, xh_ref, tsrc_ref, tdst_ref)


def _postpre(h, a0, a1, d0, d1, eexp, bg, lng, lnb, wg, acat):
    return pl.pallas_call(
        _postpre_body,
        grid=(NP // RB,),
        in_specs=_COMBINE_SPECS + [_full((HID, HID)), _full((HID, 32))],
        out_specs=[_rows(HID), _rows(HID), _rows(16), _rows(16)],
        out_shape=[
            jax.ShapeDtypeStruct((NP, HID), f32),
            jax.ShapeDtypeStruct((NP, HID), f32),
            jax.ShapeDtypeStruct((NP, 16), f32),
            jax.ShapeDtypeStruct((NP, 16), f32),
        ],
    )(h, a0, a1, d0, d1, eexp, bg, lng, lnb, wg, acat)


def _postepi_body(h_ref, a0_ref, a1_ref, d0_ref, d1_ref, eexp_ref, bg_ref,
                  lng_ref, lnb_ref, w1_ref, b1_ref, w2_ref, b2_ref, o_ref):
    h = _combine(h_ref, a0_ref, a1_ref, d0_ref, d1_ref, eexp_ref, bg_ref,
                 lng_ref, lnb_ref)
    o1 = jax.nn.relu(jnp.dot(h, w1_ref[...],
                             preferred_element_type=f32) + b1_ref[...])
    o_ref[...] = jax.nn.relu(jnp.dot(o1, w2_ref[...],
                                     preferred_element_type=f32) + b2_ref[...])


def _postepi(h, a0, a1, d0, d1, eexp, bg, lng, lnb, w1, b1, w2p, b2p):
    return pl.pallas_call(
        _postepi_body,
        grid=(NP // RB,),
        in_specs=_COMBINE_SPECS + [
            _full((HID, HID)), _full((1, HID)),
            _full((HID, HID)), _full((1, HID)),
        ],
        out_specs=_rows(HID),
        out_shape=jax.ShapeDtypeStruct((NP, HID), f32),
    )(h, a0, a1, d0, d1, eexp, bg, lng, lnb, w1, b1, w2p, b2p)


def _edge_body(xh_hbm, tsrc_hbm, tdst_hbm, ae_hbm, src_hbm, dst_hbm, gv_hbm,
               acc0_hbm, acc1_hbm, den0_hbm, den1_hbm,
               acc_sh, den_sh, gvbuf,
               sidx0, sidx1, didx0, didx1, bsrc0, bsrc1, bdst0, bdst1,
               bae0, bae1, bex0, bex1, rows0, rows1,
               semi0, semi1, sems0, sems1, semd0, semd1, sema0, sema1,
               semr0, semr1, semz):
    core = lax.axis_index("c")
    sid = lax.axis_index("s")
    wid = sid * 2 + core

    sidxb = (sidx0, sidx1)
    didxb = (didx0, didx1)
    bsrc = (bsrc0, bsrc1)
    bdst = (bdst0, bdst1)
    bae = (bae0, bae1)
    bex = (bex0, bex1)
    rows = (rows0, rows1)
    semi = (semi0, semi1)
    sems = (sems0, sems1)
    semd = (semd0, semd1)
    sema = (sema0, sema1)
    semr = (semr0, semr1)

    pltpu.sync_copy(gv_hbm, gvbuf)
    gvreg = gvbuf[...]

    zero16 = jnp.zeros((16,), f32)

    @pl.loop(0, CH)
    def _(r):
        bex0[r, :] = zero16

        @pl.loop(0, 8)
        def _(g):
            rows0[r, pl.ds(g * 16, 16)] = zero16

    rbase = sid * ROWS_PER_SUB
    PART = ROWS_PER_SUB - 6 * CH

    zcopies = []
    for j in range(6):
        zcopies.append(pltpu.async_copy(
            rows0, acc_sh.at[pl.ds(rbase + j * CH, CH)], semz))
        zcopies.append(pltpu.async_copy(
            bex0, den_sh.at[pl.ds(rbase + j * CH, CH)], semz))
    zcopies.append(pltpu.async_copy(
        rows0.at[pl.ds(0, PART)],
        acc_sh.at[pl.ds(rbase + 6 * CH, PART)], semz))
    zcopies.append(pltpu.async_copy(
        bex0.at[pl.ds(0, PART)],
        den_sh.at[pl.ds(rbase + 6 * CH, PART)], semz))
    for cp in zcopies:
        cp.wait()

    plsc.subcore_barrier()

    lane_g = [jnp.full((16,), g, jnp.int32) for g in range(8)]

    def issue_idx(i, p):
        cid = wid + i * NWORK

        @pl.when(cid < NCHUNK)
        def _():
            pltpu.async_copy(src_hbm.at[pl.ds(cid * CH, CH)], sidxb[p],
                             semi[p])
            pltpu.async_copy(dst_hbm.at[pl.ds(cid * CH, CH)], didxb[p],
                             semi[p])

    def issue_gathers(i, p):
        cid = wid + i * NWORK

        @pl.when(cid < NCHUNK)
        def _():
            pltpu.make_async_copy(src_hbm.at[pl.ds(cid * CH, CH)],
                                  sidxb[p], semi[p]).wait()
            pltpu.make_async_copy(dst_hbm.at[pl.ds(cid * CH, CH)],
                                  didxb[p], semi[p]).wait()
            pltpu.async_copy(xh_hbm.at[sidxb[p]], rows[p], semr[p])
            pltpu.async_copy(tsrc_hbm.at[sidxb[p]], bsrc[p], sems[p])
            pltpu.async_copy(tdst_hbm.at[didxb[p]], bdst[p], semd[p])
            pltpu.async_copy(ae_hbm.at[pl.ds(cid * CH * 16, CH * 16)],
                             bae[p], sema[p])

    def step(i, p):
        issue_gathers(i + 1, 1 - p)
        cid = wid + i * NWORK

        @pl.when(cid < NCHUNK)
        def _():
            pltpu.make_async_copy(tsrc_hbm.at[sidxb[p]], bsrc[p],
                                  sems[p]).wait()
            pltpu.make_async_copy(tdst_hbm.at[didxb[p]], bdst[p],
                                  semd[p]).wait()
            pltpu.make_async_copy(ae_hbm.at[pl.ds(cid * CH * 16, CH * 16)],
                                  bae[p], sema[p]).wait()

            @plsc.parallel_loop(0, CH, unroll=2)
            def _(k):
                vs = bsrc[p][k, :]
                vd = bdst[p][k, :]
                va = bae[p][pl.ds(k * 16, 16)]
                raw = vs + vd + va
                alpha = jnp.maximum(raw, 0.2 * raw)
                tt = vd + gvreg
                m = jnp.maximum(tt, 0.2 * tt)
                bex[p][k, :] = jnp.exp(alpha - m)

            pltpu.sync_copy(bex[p], den_sh.at[didxb[p]], add=True)
            pltpu.make_async_copy(xh_hbm.at[sidxb[p]], rows[p],
                                  semr[p]).wait()

            @plsc.parallel_loop(0, CH, unroll=4)
            def _(k):
                kvec = jnp.full((16,), k, jnp.int32)
                for g in range(8):
                    sc = plsc.load_gather(bex[p], [kvec, lane_g[g]])
                    rows[p][k, pl.ds(g * 16, 16)] = (
                        rows[p][k, pl.ds(g * 16, 16)] * sc)

            pltpu.sync_copy(rows[p], acc_sh.at[didxb[p]], add=True)
            issue_idx(i + 2, p)

    issue_idx(0, 0)
    issue_gathers(0, 0)
    issue_idx(1, 1)

    @pl.loop(0, (TPW + 1) // 2)
    def _(u):
        step(2 * u, 0)
        step(2 * u + 1, 1)

    plsc.subcore_barrier()

    @pl.when(core == 0)
    def _():
        dcopies = []
        for j in range(6):
            ro = rbase + j * CH
            dcopies.append(pltpu.async_copy(
                acc_sh.at[pl.ds(ro, CH)], acc0_hbm.at[pl.ds(ro, CH)], semz))
            dcopies.append(pltpu.async_copy(
                den_sh.at[pl.ds(ro, CH)], den0_hbm.at[pl.ds(ro, CH)], semz))
        ro2 = rbase + 6 * CH
        dcopies.append(pltpu.async_copy(
            acc_sh.at[pl.ds(ro2, PART)], acc0_hbm.at[pl.ds(ro2, PART)], semz))
        dcopies.append(pltpu.async_copy(
            den_sh.at[pl.ds(ro2, PART)], den0_hbm.at[pl.ds(ro2, PART)], semz))
        for cp in dcopies:
            cp.wait()

    @pl.when(core == 1)
    def _():
        dcopies = []
        for j in range(6):
            ro = rbase + j * CH
            dcopies.append(pltpu.async_copy(
                acc_sh.at[pl.ds(ro, CH)], acc1_hbm.at[pl.ds(ro, CH)], semz))
            dcopies.append(pltpu.async_copy(
                den_sh.at[pl.ds(ro, CH)], den1_hbm.at[pl.ds(ro, CH)], semz))
        ro2 = rbase + 6 * CH
        dcopies.append(pltpu.async_copy(
            acc_sh.at[pl.ds(ro2, PART)], acc1_hbm.at[pl.ds(ro2, PART)], semz))
        dcopies.append(pltpu.async_copy(
            den_sh.at[pl.ds(ro2, PART)], den1_hbm.at[pl.ds(ro2, PART)], semz))
        for cp in dcopies:
            cp.wait()


def _edge_kernel(xh, tsrc, tdst, ae_l, srcp, dstp, gv):
    mesh = plsc.VectorSubcoreMesh(core_axis_name="c", subcore_axis_name="s")
    kfn = pl.kernel(
        _edge_body,
        out_type=[
            jax.ShapeDtypeStruct((NP, HID), f32),
            jax.ShapeDtypeStruct((NP, HID), f32),
            jax.ShapeDtypeStruct((NP, 16), f32),
            jax.ShapeDtypeStruct((NP, 16), f32),
        ],
        mesh=mesh,
        scratch_types=[
            pltpu.VMEM_SHARED((NP, HID), f32),
            pltpu.VMEM_SHARED((NP, 16), f32),
            pltpu.VMEM((16,), f32),
            pltpu.VMEM((CH,), jnp.int32),
            pltpu.VMEM((CH,), jnp.int32),
            pltpu.VMEM((CH,), jnp.int32),
            pltpu.VMEM((CH,), jnp.int32),
            pltpu.VMEM((CH, 16), f32),
            pltpu.VMEM((CH, 16), f32),
            pltpu.VMEM((CH, 16), f32),
            pltpu.VMEM((CH, 16), f32),
            pltpu.VMEM((CH * 16,), f32),
            pltpu.VMEM((CH * 16,), f32),
            pltpu.VMEM((CH, 16), f32),
            pltpu.VMEM((CH, 16), f32),
            pltpu.VMEM((CH, HID), f32),
            pltpu.VMEM((CH, HID), f32),
        ] + [pltpu.SemaphoreType.DMA] * 11,
        compiler_params=pltpu.CompilerParams(use_tc_tiling_on_sc=False,
                                             needs_layout_passes=False),
    )
    return kfn(xh, tsrc, tdst, ae_l, srcp, dstp, gv)


def kernel(x, edge_index, edge_attr, W_in, b_in, ln_in_g, ln_in_b, W_gat,
           b_gat, att_src, att_dst, W_edge, att_edge, ln_g, ln_b, W_h1, b_h1,
           W_h2, b_h2):
    eye8 = jnp.eye(H, dtype=f32)
    asf = (att_src[:, :, :, None] * eye8[:, None, :]).reshape(LAYERS, HID, H)
    adf = (att_dst[:, :, :, None] * eye8[:, None, :]).reshape(LAYERS, HID, H)
    acat = jnp.concatenate([asf, asf, adf, adf], axis=-1)
    me = jnp.einsum('ldhc,lhc->ldh',
                    W_edge.reshape(LAYERS, EDGE_DIM, H, C), att_edge)
    medup = jnp.concatenate([me, me], axis=-1)
    klist = [jnp.kron(eye8, medup[l]) for l in range(LAYERS)]
    eexp = jnp.concatenate(
        [jnp.repeat(eye8, 16, axis=1), jnp.zeros((8, HID), f32)], axis=0)
    w2p = jnp.zeros((HID, HID), f32).at[:, :OUT_DIM].set(W_h2)
    b2p = jnp.zeros((1, HID), f32).at[0, :OUT_DIM].set(b_h2)

    xp = jnp.concatenate([x, jnp.zeros((NP - N, HID), f32)], axis=0)
    fill = jnp.full((EP - E,), NP - 1, edge_index.dtype)
    srcp = jnp.concatenate([edge_index[0], fill])
    dstp = jnp.concatenate([edge_index[1], fill])
    eap = jnp.concatenate([edge_attr, jnp.zeros((EP - E, EDGE_DIM), f32)],
                          axis=0)

    ea_r = eap.reshape(EP // 8, 8 * EDGE_DIM)
    ae_out = _ae_kernel(ea_r, klist)
    ae_list = ae_out[:LAYERS]
    gmax_e = [jnp.max(ae_out[LAYERS + l], axis=(0, 1)).reshape(8, 16)
              .max(axis=0) for l in range(LAYERS)]

    h, xh, tsrc, tdst = _prologue(xp, W_in, b_in.reshape(1, HID),
                                  ln_in_g.reshape(1, HID),
                                  ln_in_b.reshape(1, HID), W_gat[0], acat[0])

    for l in range(LAYERS):
        gv = jnp.max(tsrc[:N], axis=0) + gmax_e[l]
        ae_flat = ae_list[l].reshape(EP * 16)
        a0, a1, d0, d1 = _edge_kernel(xh, tsrc, tdst, ae_flat, srcp, dstp, gv)
        bg = b_gat[l].reshape(1, HID)
        lg = ln_g[l].reshape(1, HID)
        lb = ln_b[l].reshape(1, HID)
        if l < LAYERS - 1:
            h, xh, tsrc, tdst = _postpre(h, a0, a1, d0, d1, eexp, bg, lg, lb,
                                         W_gat[l + 1], acat[l + 1])
        else:
            out = _postepi(h, a0, a1, d0, d1, eexp, bg, lg, lb,
                           W_h1, b_h1.reshape(1, HID), w2p, b2p)

    return out[:N, :OUT_DIM]


# speedup vs baseline: 1.0980x; 1.0980x over previous
import functools

import jax
import jax.numpy as jnp
from jax import lax
from jax.experimental import pallas as pl
from jax.experimental.pallas import tpu as pltpu
from jax.experimental.pallas import tpu_sc as plsc

N = 10000
NP = 10240
E = 320000
EP = 322560
HID = 128
H = 8
C = 16
EDGE_DIM = 4
LAYERS = 6
OUT_DIM = 3

RB = 1024
CH = 96
NCHUNK = EP // CH
NWORK = 32
TPW = NCHUNK // NWORK
ROWS_PER_SUB = NP // 16

f32 = jnp.float32


def _ln(t, g, b):
    mu = jnp.mean(t, axis=-1, keepdims=True)
    var = jnp.mean((t - mu) ** 2, axis=-1, keepdims=True)
    return (t - mu) / jnp.sqrt(var + 1e-5) * g + b


def _full(shape):
    return pl.BlockSpec(shape, lambda i: tuple(0 for _ in shape))


def _rows(w):
    return pl.BlockSpec((RB, w), lambda i: (i, 0))


def _pre_part(h, w_ref, acat_ref, xh_ref, tsrc_ref, tdst_ref):
    xh = jnp.dot(h, w_ref[...], preferred_element_type=f32)
    al = jnp.dot(xh, acat_ref[...], preferred_element_type=f32)
    xh_ref[...] = xh
    tsrc_ref[...] = al[:, :16]
    tdst_ref[...] = al[:, 16:]


def _prologue_body(x_ref, w_ref, b_ref, g_ref, bb_ref, wg_ref, acat_ref,
                   o_ref, xh_ref, tsrc_ref, tdst_ref):
    h = jax.nn.relu(jnp.dot(x_ref[...], w_ref[...],
                            preferred_element_type=f32) + b_ref[...])
    h = _ln(h, g_ref[...], bb_ref[...])
    o_ref[...] = h
    _pre_part(h, wg_ref, acat_ref, xh_ref, tsrc_ref, tdst_ref)


def _prologue(x, w, b, g, bb, wg, acat):
    return pl.pallas_call(
        _prologue_body,
        grid=(NP // RB,),
        in_specs=[
            _rows(HID), _full((HID, HID)), _full((1, HID)), _full((1, HID)),
            _full((1, HID)), _full((HID, HID)), _full((HID, 32)),
        ],
        out_specs=[_rows(HID), _rows(HID), _rows(16), _rows(16)],
        out_shape=[
            jax.ShapeDtypeStruct((NP, HID), f32),
            jax.ShapeDtypeStruct((NP, HID), f32),
            jax.ShapeDtypeStruct((NP, 16), f32),
            jax.ShapeDtypeStruct((NP, 16), f32),
        ],
    )(x, w, b, g, bb, wg, acat)


def _ae_body(ea_ref, *refs):
    k_refs = refs[:LAYERS]
    o_refs = refs[LAYERS:2 * LAYERS]
    m_refs = refs[2 * LAYERS:]
    ea = ea_ref[...]
    for l in range(LAYERS):
        r = jnp.dot(ea, k_refs[l][...], preferred_element_type=f32)
        o_refs[l][...] = r
        m_refs[l][...] = jnp.max(r, axis=0).reshape(1, 1, HID)


def _ae_kernel(ea_r, klist):
    EB8 = EP // 8 // 24
    return pl.pallas_call(
        _ae_body,
        grid=(24,),
        in_specs=[pl.BlockSpec((EB8, 32), lambda i: (i, 0))] +
                 [_full((32, HID)) for _ in range(LAYERS)],
        out_specs=[pl.BlockSpec((EB8, HID), lambda i: (i, 0))
                   for _ in range(LAYERS)] +
                  [pl.BlockSpec((1, 1, HID), lambda i: (i, 0, 0))
                   for _ in range(LAYERS)],
        out_shape=[jax.ShapeDtypeStruct((EP // 8, HID), f32)
                   for _ in range(LAYERS)] +
                  [jax.ShapeDtypeStruct((24, 1, HID), f32)
                   for _ in range(LAYERS)],
    )(ea_r, *klist)


def _combine(h_ref, a0_ref, a1_ref, d0_ref, d1_ref, eexp_ref, bg_ref,
             lng_ref, lnb_ref):
    acc = a0_ref[...] + a1_ref[...]
    den = d0_ref[...] + d1_ref[...]
    denx = jnp.dot(den, eexp_ref[...], preferred_element_type=f32)
    g = acc / (denx + 1e-16) + bg_ref[...]
    return _ln(h_ref[...] + g, lng_ref[...], lnb_ref[...])


_COMBINE_SPECS = [
    _rows(HID), _rows(HID), _rows(HID), _rows(16), _rows(16),
    _full((16, HID)), _full((1, HID)), _full((1, HID)), _full((1, HID)),
]


def _postpre_body(h_ref, a0_ref, a1_ref, d0_ref, d1_ref, eexp_ref, bg_ref,
                  lng_ref, lnb_ref, wg_ref, acat_ref,
                  o_ref, xh_ref, tsrc_ref, tdst_ref):
    h = _combine(h_ref, a0_ref, a1_ref, d0_ref, d1_ref, eexp_ref, bg_ref,
                 lng_ref, lnb_ref)
    o_ref[...] = h
    _pre_part(h, wg_ref, acat_ref, xh_ref, tsrc_ref, tdst_ref)


def _postpre(h, a0, a1, d0, d1, eexp, bg, lng, lnb, wg, acat):
    return pl.pallas_call(
        _postpre_body,
        grid=(NP // RB,),
        in_specs=_COMBINE_SPECS + [_full((HID, HID)), _full((HID, 32))],
        out_specs=[_rows(HID), _rows(HID), _rows(16), _rows(16)],
        out_shape=[
            jax.ShapeDtypeStruct((NP, HID), f32),
            jax.ShapeDtypeStruct((NP, HID), f32),
            jax.ShapeDtypeStruct((NP, 16), f32),
            jax.ShapeDtypeStruct((NP, 16), f32),
        ],
    )(h, a0, a1, d0, d1, eexp, bg, lng, lnb, wg, acat)


def _postepi_body(h_ref, a0_ref, a1_ref, d0_ref, d1_ref, eexp_ref, bg_ref,
                  lng_ref, lnb_ref, w1_ref, b1_ref, w2_ref, b2_ref, o_ref):
    h = _combine(h_ref, a0_ref, a1_ref, d0_ref, d1_ref, eexp_ref, bg_ref,
                 lng_ref, lnb_ref)
    o1 = jax.nn.relu(jnp.dot(h, w1_ref[...],
                             preferred_element_type=f32) + b1_ref[...])
    o_ref[...] = jax.nn.relu(jnp.dot(o1, w2_ref[...],
                                     preferred_element_type=f32) + b2_ref[...])


def _postepi(h, a0, a1, d0, d1, eexp, bg, lng, lnb, w1, b1, w2p, b2p):
    return pl.pallas_call(
        _postepi_body,
        grid=(NP // RB,),
        in_specs=_COMBINE_SPECS + [
            _full((HID, HID)), _full((1, HID)),
            _full((HID, HID)), _full((1, HID)),
        ],
        out_specs=_rows(HID),
        out_shape=jax.ShapeDtypeStruct((NP, HID), f32),
    )(h, a0, a1, d0, d1, eexp, bg, lng, lnb, w1, b1, w2p, b2p)


def _edge_body(xh_hbm, tsrc_hbm, tdst_hbm, ae_hbm, src_hbm, dst_hbm, gv_hbm,
               acc0_hbm, acc1_hbm, den0_hbm, den1_hbm,
               acc_sh, den_sh, gvbuf,
               sidx0, sidx1, didx0, didx1, bsrc0, bsrc1, bdst0, bdst1,
               bae0, bae1, bex0, bex1, rows0, rows1,
               semi0, semi1, sems0, sems1, semd0, semd1, sema0, sema1,
               semr0, semr1, semz):
    core = lax.axis_index("c")
    sid = lax.axis_index("s")
    wid = sid * 2 + core

    sidxb = (sidx0, sidx1)
    didxb = (didx0, didx1)
    bsrc = (bsrc0, bsrc1)
    bdst = (bdst0, bdst1)
    bae = (bae0, bae1)
    bex = (bex0, bex1)
    rows = (rows0, rows1)
    semi = (semi0, semi1)
    sems = (sems0, sems1)
    semd = (semd0, semd1)
    sema = (sema0, sema1)
    semr = (semr0, semr1)

    pltpu.sync_copy(gv_hbm, gvbuf)
    gvreg = gvbuf[...]

    zero16 = jnp.zeros((16,), f32)

    @pl.loop(0, CH)
    def _(r):
        bex0[r, :] = zero16

        @pl.loop(0, 8)
        def _(g):
            rows0[r, pl.ds(g * 16, 16)] = zero16

    rbase = sid * ROWS_PER_SUB
    PART = ROWS_PER_SUB - 6 * CH

    zcopies = []
    for j in range(6):
        zcopies.append(pltpu.async_copy(
            rows0, acc_sh.at[pl.ds(rbase + j * CH, CH)], semz))
        zcopies.append(pltpu.async_copy(
            bex0, den_sh.at[pl.ds(rbase + j * CH, CH)], semz))
    zcopies.append(pltpu.async_copy(
        rows0.at[pl.ds(0, PART)],
        acc_sh.at[pl.ds(rbase + 6 * CH, PART)], semz))
    zcopies.append(pltpu.async_copy(
        bex0.at[pl.ds(0, PART)],
        den_sh.at[pl.ds(rbase + 6 * CH, PART)], semz))
    for cp in zcopies:
        cp.wait()

    plsc.subcore_barrier()

    lane_g = [jnp.full((16,), g, jnp.int32) for g in range(8)]

    def issue_idx(i, p):
        cid = wid + i * NWORK

        @pl.when(cid < NCHUNK)
        def _():
            pltpu.async_copy(src_hbm.at[pl.ds(cid * CH, CH)], sidxb[p],
                             semi[p])
            pltpu.async_copy(dst_hbm.at[pl.ds(cid * CH, CH)], didxb[p],
                             semi[p])

    def issue_gathers(i, p):
        cid = wid + i * NWORK

        @pl.when(cid < NCHUNK)
        def _():
            pltpu.make_async_copy(src_hbm.at[pl.ds(cid * CH, CH)],
                                  sidxb[p], semi[p]).wait()
            pltpu.make_async_copy(dst_hbm.at[pl.ds(cid * CH, CH)],
                                  didxb[p], semi[p]).wait()
            pltpu.async_copy(xh_hbm.at[sidxb[p]], rows[p], semr[p])
            pltpu.async_copy(tsrc_hbm.at[sidxb[p]], bsrc[p], sems[p])
            pltpu.async_copy(tdst_hbm.at[didxb[p]], bdst[p], semd[p])
            pltpu.async_copy(ae_hbm.at[pl.ds(cid * CH * 16, CH * 16)],
                             bae[p], sema[p])

    def step(i, p):
        issue_gathers(i + 1, 1 - p)
        cid = wid + i * NWORK

        @pl.when(cid < NCHUNK)
        def _():
            pltpu.make_async_copy(tsrc_hbm.at[sidxb[p]], bsrc[p],
                                  sems[p]).wait()
            pltpu.make_async_copy(tdst_hbm.at[didxb[p]], bdst[p],
                                  semd[p]).wait()
            pltpu.make_async_copy(ae_hbm.at[pl.ds(cid * CH * 16, CH * 16)],
                                  bae[p], sema[p]).wait()

            @plsc.parallel_loop(0, CH, unroll=2)
            def _(k):
                vs = bsrc[p][k, :]
                vd = bdst[p][k, :]
                va = bae[p][pl.ds(k * 16, 16)]
                raw = vs + vd + va
                alpha = jnp.maximum(raw, 0.2 * raw)
                tt = vd + gvreg
                m = jnp.maximum(tt, 0.2 * tt)
                bex[p][k, :] = jnp.exp(alpha - m)

            pltpu.sync_copy(bex[p], den_sh.at[didxb[p]], add=True)
            pltpu.make_async_copy(xh_hbm.at[sidxb[p]], rows[p],
                                  semr[p]).wait()

            @plsc.parallel_loop(0, CH, unroll=2)
            def _(k):
                ex = bex[p][k, :]
                for g in range(8):
                    sc = jnp.full((16,), ex[g], f32)
                    rows[p][k, pl.ds(g * 16, 16)] = (
                        rows[p][k, pl.ds(g * 16, 16)] * sc)

            pltpu.sync_copy(rows[p], acc_sh.at[didxb[p]], add=True)
            issue_idx(i + 2, p)

    issue_idx(0, 0)
    issue_gathers(0, 0)
    issue_idx(1, 1)

    @pl.loop(0, (TPW + 1) // 2)
    def _(u):
        step(2 * u, 0)
        step(2 * u + 1, 1)

    plsc.subcore_barrier()

    @pl.when(core == 0)
    def _():
        dcopies = []
        for j in range(6):
            ro = rbase + j * CH
            dcopies.append(pltpu.async_copy(
                acc_sh.at[pl.ds(ro, CH)], acc0_hbm.at[pl.ds(ro, CH)], semz))
            dcopies.append(pltpu.async_copy(
                den_sh.at[pl.ds(ro, CH)], den0_hbm.at[pl.ds(ro, CH)], semz))
        ro2 = rbase + 6 * CH
        dcopies.append(pltpu.async_copy(
            acc_sh.at[pl.ds(ro2, PART)], acc0_hbm.at[pl.ds(ro2, PART)], semz))
        dcopies.append(pltpu.async_copy(
            den_sh.at[pl.ds(ro2, PART)], den0_hbm.at[pl.ds(ro2, PART)], semz))
        for cp in dcopies:
            cp.wait()

    @pl.when(core == 1)
    def _():
        dcopies = []
        for j in range(6):
            ro = rbase + j * CH
            dcopies.append(pltpu.async_copy(
                acc_sh.at[pl.ds(ro, CH)], acc1_hbm.at[pl.ds(ro, CH)], semz))
            dcopies.append(pltpu.async_copy(
                den_sh.at[pl.ds(ro, CH)], den1_hbm.at[pl.ds(ro, CH)], semz))
        ro2 = rbase + 6 * CH
        dcopies.append(pltpu.async_copy(
            acc_sh.at[pl.ds(ro2, PART)], acc1_hbm.at[pl.ds(ro2, PART)], semz))
        dcopies.append(pltpu.async_copy(
            den_sh.at[pl.ds(ro2, PART)], den1_hbm.at[pl.ds(ro2, PART)], semz))
        for cp in dcopies:
            cp.wait()


def _edge_kernel(xh, tsrc, tdst, ae_l, srcp, dstp, gv):
    mesh = plsc.VectorSubcoreMesh(core_axis_name="c", subcore_axis_name="s")
    kfn = pl.kernel(
        _edge_body,
        out_type=[
            jax.ShapeDtypeStruct((NP, HID), f32),
            jax.ShapeDtypeStruct((NP, HID), f32),
            jax.ShapeDtypeStruct((NP, 16), f32),
            jax.ShapeDtypeStruct((NP, 16), f32),
        ],
        mesh=mesh,
        scratch_types=[
            pltpu.VMEM_SHARED((NP, HID), f32),
            pltpu.VMEM_SHARED((NP, 16), f32),
            pltpu.VMEM((16,), f32),
            pltpu.VMEM((CH,), jnp.int32),
            pltpu.VMEM((CH,), jnp.int32),
            pltpu.VMEM((CH,), jnp.int32),
            pltpu.VMEM((CH,), jnp.int32),
            pltpu.VMEM((CH, 16), f32),
            pltpu.VMEM((CH, 16), f32),
            pltpu.VMEM((CH, 16), f32),
            pltpu.VMEM((CH, 16), f32),
            pltpu.VMEM((CH * 16,), f32),
            pltpu.VMEM((CH * 16,), f32),
            pltpu.VMEM((CH, 16), f32),
            pltpu.VMEM((CH, 16), f32),
            pltpu.VMEM((CH, HID), f32),
            pltpu.VMEM((CH, HID), f32),
        ] + [pltpu.SemaphoreType.DMA] * 11,
        compiler_params=pltpu.CompilerParams(use_tc_tiling_on_sc=False,
                                             needs_layout_passes=False),
    )
    return kfn(xh, tsrc, tdst, ae_l, srcp, dstp, gv)


def kernel(x, edge_index, edge_attr, W_in, b_in, ln_in_g, ln_in_b, W_gat,
           b_gat, att_src, att_dst, W_edge, att_edge, ln_g, ln_b, W_h1, b_h1,
           W_h2, b_h2):
    eye8 = jnp.eye(H, dtype=f32)
    asf = (att_src[:, :, :, None] * eye8[:, None, :]).reshape(LAYERS, HID, H)
    adf = (att_dst[:, :, :, None] * eye8[:, None, :]).reshape(LAYERS, HID, H)
    acat = jnp.concatenate([asf, asf, adf, adf], axis=-1)
    me = jnp.einsum('ldhc,lhc->ldh',
                    W_edge.reshape(LAYERS, EDGE_DIM, H, C), att_edge)
    medup = jnp.concatenate([me, me], axis=-1)
    klist = [jnp.kron(eye8, medup[l]) for l in range(LAYERS)]
    eexp = jnp.concatenate(
        [jnp.repeat(eye8, 16, axis=1), jnp.zeros((8, HID), f32)], axis=0)
    w2p = jnp.zeros((HID, HID), f32).at[:, :OUT_DIM].set(W_h2)
    b2p = jnp.zeros((1, HID), f32).at[0, :OUT_DIM].set(b_h2)

    xp = jnp.concatenate([x, jnp.zeros((NP - N, HID), f32)], axis=0)
    fill = jnp.full((EP - E,), NP - 1, edge_index.dtype)
    srcp = jnp.concatenate([edge_index[0], fill])
    dstp = jnp.concatenate([edge_index[1], fill])
    eap = jnp.concatenate([edge_attr, jnp.zeros((EP - E, EDGE_DIM), f32)],
                          axis=0)

    ea_r = eap.reshape(EP // 8, 8 * EDGE_DIM)
    ae_out = _ae_kernel(ea_r, klist)
    ae_list = ae_out[:LAYERS]
    gmax_e = [jnp.max(ae_out[LAYERS + l], axis=(0, 1)).reshape(8, 16)
              .max(axis=0) for l in range(LAYERS)]

    h, xh, tsrc, tdst = _prologue(xp, W_in, b_in.reshape(1, HID),
                                  ln_in_g.reshape(1, HID),
                                  ln_in_b.reshape(1, HID), W_gat[0], acat[0])

    for l in range(LAYERS):
        gv = jnp.max(tsrc[:N], axis=0) + gmax_e[l]
        ae_flat = ae_list[l].reshape(EP * 16)
        a0, a1, d0, d1 = _edge_kernel(xh, tsrc, tdst, ae_flat, srcp, dstp, gv)
        bg = b_gat[l].reshape(1, HID)
        lg = ln_g[l].reshape(1, HID)
        lb = ln_b[l].reshape(1, HID)
        if l < LAYERS - 1:
            h, xh, tsrc, tdst = _postpre(h, a0, a1, d0, d1, eexp, bg, lg, lb,
                                         W_gat[l + 1], acat[l + 1])
        else:
            out = _postepi(h, a0, a1, d0, d1, eexp, bg, lg, lb,
                           W_h1, b_h1.reshape(1, HID), w2p, b2p)

    return out[:N, :OUT_DIM]


# speedup vs baseline: 1.2444x; 1.1333x over previous
import functools

import jax
import jax.numpy as jnp
from jax import lax
from jax.experimental import pallas as pl
from jax.experimental.pallas import tpu as pltpu
from jax.experimental.pallas import tpu_sc as plsc

N = 10000
NP = 10240
E = 320000
EP = 322560
HID = 128
H = 8
C = 16
EDGE_DIM = 4
LAYERS = 6
OUT_DIM = 3

RB = 1024
CH = 96
NCHUNK = EP // CH
NWORK = 32
TPW = NCHUNK // NWORK
ROWS_PER_SUB = NP // 16

f32 = jnp.float32


def _ln(t, g, b):
    mu = jnp.mean(t, axis=-1, keepdims=True)
    var = jnp.mean((t - mu) ** 2, axis=-1, keepdims=True)
    return (t - mu) / jnp.sqrt(var + 1e-5) * g + b


def _full(shape):
    return pl.BlockSpec(shape, lambda i: tuple(0 for _ in shape))


def _rows(w):
    return pl.BlockSpec((RB, w), lambda i: (i, 0))


def _pre_part(h, w_ref, acat_ref, xhb_ref, tsrc_ref, tdst_ref):
    xh = jnp.dot(h, w_ref[...], preferred_element_type=f32)
    al = jnp.dot(xh, acat_ref[...], preferred_element_type=f32)
    au = lax.bitcast_convert_type(xh[:, :64].astype(jnp.bfloat16),
                                  jnp.uint16).astype(jnp.uint32)
    bu = lax.bitcast_convert_type(xh[:, 64:].astype(jnp.bfloat16),
                                  jnp.uint16).astype(jnp.uint32)
    xhb_ref[...] = lax.bitcast_convert_type(au | (bu << 16), f32)
    tsrc_ref[...] = al[:, :16]
    tdst_ref[...] = al[:, 16:]


def _prologue_body(x_ref, w_ref, b_ref, g_ref, bb_ref, wg_ref, acat_ref,
                   o_ref, xh_ref, tsrc_ref, tdst_ref):
    h = jax.nn.relu(jnp.dot(x_ref[...], w_ref[...],
                            preferred_element_type=f32) + b_ref[...])
    h = _ln(h, g_ref[...], bb_ref[...])
    o_ref[...] = h
    _pre_part(h, wg_ref, acat_ref, xh_ref, tsrc_ref, tdst_ref)


def _prologue(x, w, b, g, bb, wg, acat):
    return pl.pallas_call(
        _prologue_body,
        grid=(NP // RB,),
        in_specs=[
            _rows(HID), _full((HID, HID)), _full((1, HID)), _full((1, HID)),
            _full((1, HID)), _full((HID, HID)), _full((HID, 32)),
        ],
        out_specs=[_rows(HID), _rows(64), _rows(16), _rows(16)],
        out_shape=[
            jax.ShapeDtypeStruct((NP, HID), f32),
            jax.ShapeDtypeStruct((NP, 64), f32),
            jax.ShapeDtypeStruct((NP, 16), f32),
            jax.ShapeDtypeStruct((NP, 16), f32),
        ],
    )(x, w, b, g, bb, wg, acat)


def _ae_body(ea_ref, *refs):
    k_refs = refs[:LAYERS]
    o_refs = refs[LAYERS:2 * LAYERS]
    m_refs = refs[2 * LAYERS:]
    ea = ea_ref[...]
    for l in range(LAYERS):
        r = jnp.dot(ea, k_refs[l][...], preferred_element_type=f32)
        o_refs[l][...] = r
        m_refs[l][...] = jnp.max(r, axis=0).reshape(1, 1, HID)


def _ae_kernel(ea_r, klist):
    EB8 = EP // 8 // 24
    return pl.pallas_call(
        _ae_body,
        grid=(24,),
        in_specs=[pl.BlockSpec((EB8, 32), lambda i: (i, 0))] +
                 [_full((32, HID)) for _ in range(LAYERS)],
        out_specs=[pl.BlockSpec((EB8, HID), lambda i: (i, 0))
                   for _ in range(LAYERS)] +
                  [pl.BlockSpec((1, 1, HID), lambda i: (i, 0, 0))
                   for _ in range(LAYERS)],
        out_shape=[jax.ShapeDtypeStruct((EP // 8, HID), f32)
                   for _ in range(LAYERS)] +
                  [jax.ShapeDtypeStruct((24, 1, HID), f32)
                   for _ in range(LAYERS)],
    )(ea_r, *klist)


def _combine(h_ref, a0_ref, a1_ref, d0_ref, d1_ref, eexp_ref, bg_ref,
             lng_ref, lnb_ref):
    acc = a0_ref[...] + a1_ref[...]
    den = d0_ref[...] + d1_ref[...]
    denx = jnp.dot(den, eexp_ref[...], preferred_element_type=f32)
    g = acc / (denx + 1e-16) + bg_ref[...]
    return _ln(h_ref[...] + g, lng_ref[...], lnb_ref[...])


_COMBINE_SPECS = [
    _rows(HID), _rows(HID), _rows(HID), _rows(16), _rows(16),
    _full((16, HID)), _full((1, HID)), _full((1, HID)), _full((1, HID)),
]


def _postpre_body(h_ref, a0_ref, a1_ref, d0_ref, d1_ref, eexp_ref, bg_ref,
                  lng_ref, lnb_ref, wg_ref, acat_ref,
                  o_ref, xh_ref, tsrc_ref, tdst_ref):
    h = _combine(h_ref, a0_ref, a1_ref, d0_ref, d1_ref, eexp_ref, bg_ref,
                 lng_ref, lnb_ref)
    o_ref[...] = h
    _pre_part(h, wg_ref, acat_ref, xh_ref, tsrc_ref, tdst_ref)


def _postpre(h, a0, a1, d0, d1, eexp, bg, lng, lnb, wg, acat):
    return pl.pallas_call(
        _postpre_body,
        grid=(NP // RB,),
        in_specs=_COMBINE_SPECS + [_full((HID, HID)), _full((HID, 32))],
        out_specs=[_rows(HID), _rows(64), _rows(16), _rows(16)],
        out_shape=[
            jax.ShapeDtypeStruct((NP, HID), f32),
            jax.ShapeDtypeStruct((NP, 64), f32),
            jax.ShapeDtypeStruct((NP, 16), f32),
            jax.ShapeDtypeStruct((NP, 16), f32),
        ],
    )(h, a0, a1, d0, d1, eexp, bg, lng, lnb, wg, acat)


def _postepi_body(h_ref, a0_ref, a1_ref, d0_ref, d1_ref, eexp_ref, bg_ref,
                  lng_ref, lnb_ref, w1_ref, b1_ref, w2_ref, b2_ref, o_ref):
    h = _combine(h_ref, a0_ref, a1_ref, d0_ref, d1_ref, eexp_ref, bg_ref,
                 lng_ref, lnb_ref)
    o1 = jax.nn.relu(jnp.dot(h, w1_ref[...],
                             preferred_element_type=f32) + b1_ref[...])
    o_ref[...] = jax.nn.relu(jnp.dot(o1, w2_ref[...],
                                     preferred_element_type=f32) + b2_ref[...])


def _postepi(h, a0, a1, d0, d1, eexp, bg, lng, lnb, w1, b1, w2p, b2p):
    return pl.pallas_call(
        _postepi_body,
        grid=(NP // RB,),
        in_specs=_COMBINE_SPECS + [
            _full((HID, HID)), _full((1, HID)),
            _full((HID, HID)), _full((1, HID)),
        ],
        out_specs=_rows(HID),
        out_shape=jax.ShapeDtypeStruct((NP, HID), f32),
    )(h, a0, a1, d0, d1, eexp, bg, lng, lnb, w1, b1, w2p, b2p)


def _edge_body(xh_hbm, tsrc_hbm, tdst_hbm, ae_hbm, src_hbm, dst_hbm, gv_hbm,
               acc0_hbm, acc1_hbm, den0_hbm, den1_hbm,
               acc_sh, den_sh, gvbuf,
               sidx0, sidx1, didx0, didx1, bsrc0, bsrc1, bdst0, bdst1,
               bae0, bae1, bex0, bex1, rows0, rows1, frows,
               semi0, semi1, sems0, sems1, semd0, semd1, sema0, sema1,
               semr0, semr1, semz):
    core = lax.axis_index("c")
    sid = lax.axis_index("s")
    wid = sid * 2 + core

    sidxb = (sidx0, sidx1)
    didxb = (didx0, didx1)
    bsrc = (bsrc0, bsrc1)
    bdst = (bdst0, bdst1)
    bae = (bae0, bae1)
    bex = (bex0, bex1)
    rows = (rows0, rows1)
    semi = (semi0, semi1)
    sems = (sems0, sems1)
    semd = (semd0, semd1)
    sema = (sema0, sema1)
    semr = (semr0, semr1)

    pltpu.sync_copy(gv_hbm, gvbuf)
    gvreg = gvbuf[...]

    zero16 = jnp.zeros((16,), f32)

    @pl.loop(0, CH)
    def _(r):
        bex0[r, :] = zero16

        @pl.loop(0, 8)
        def _(g):
            frows[r, pl.ds(g * 16, 16)] = zero16

    rbase = sid * ROWS_PER_SUB
    PART = ROWS_PER_SUB - 6 * CH

    zcopies = []
    for j in range(6):
        zcopies.append(pltpu.async_copy(
            frows, acc_sh.at[pl.ds(rbase + j * CH, CH)], semz))
        zcopies.append(pltpu.async_copy(
            bex0, den_sh.at[pl.ds(rbase + j * CH, CH)], semz))
    zcopies.append(pltpu.async_copy(
        frows.at[pl.ds(0, PART)],
        acc_sh.at[pl.ds(rbase + 6 * CH, PART)], semz))
    zcopies.append(pltpu.async_copy(
        bex0.at[pl.ds(0, PART)],
        den_sh.at[pl.ds(rbase + 6 * CH, PART)], semz))
    for cp in zcopies:
        cp.wait()

    plsc.subcore_barrier()

    lane_g = [jnp.full((16,), g, jnp.int32) for g in range(8)]

    def issue_idx(i, p):
        cid = wid + i * NWORK

        @pl.when(cid < NCHUNK)
        def _():
            pltpu.async_copy(src_hbm.at[pl.ds(cid * CH, CH)], sidxb[p],
                             semi[p])
            pltpu.async_copy(dst_hbm.at[pl.ds(cid * CH, CH)], didxb[p],
                             semi[p])

    def issue_gathers(i, p):
        cid = wid + i * NWORK

        @pl.when(cid < NCHUNK)
        def _():
            pltpu.make_async_copy(src_hbm.at[pl.ds(cid * CH, CH)],
                                  sidxb[p], semi[p]).wait()
            pltpu.make_async_copy(dst_hbm.at[pl.ds(cid * CH, CH)],
                                  didxb[p], semi[p]).wait()
            pltpu.async_copy(xh_hbm.at[sidxb[p]], rows[p], semr[p])
            pltpu.async_copy(tsrc_hbm.at[sidxb[p]], bsrc[p], sems[p])
            pltpu.async_copy(tdst_hbm.at[didxb[p]], bdst[p], semd[p])
            pltpu.async_copy(ae_hbm.at[pl.ds(cid * CH * 16, CH * 16)],
                             bae[p], sema[p])

    def step(i, p):
        issue_gathers(i + 1, 1 - p)
        cid = wid + i * NWORK

        @pl.when(cid < NCHUNK)
        def _():
            pltpu.make_async_copy(tsrc_hbm.at[sidxb[p]], bsrc[p],
                                  sems[p]).wait()
            pltpu.make_async_copy(tdst_hbm.at[didxb[p]], bdst[p],
                                  semd[p]).wait()
            pltpu.make_async_copy(ae_hbm.at[pl.ds(cid * CH * 16, CH * 16)],
                                  bae[p], sema[p]).wait()

            @plsc.parallel_loop(0, CH, unroll=2)
            def _(k):
                vs = bsrc[p][k, :]
                vd = bdst[p][k, :]
                va = bae[p][pl.ds(k * 16, 16)]
                raw = vs + vd + va
                alpha = jnp.maximum(raw, 0.2 * raw)
                tt = vd + gvreg
                m = jnp.maximum(tt, 0.2 * tt)
                bex[p][k, :] = jnp.exp(alpha - m)

            pltpu.sync_copy(bex[p], den_sh.at[didxb[p]], add=True)
            pltpu.make_async_copy(xh_hbm.at[sidxb[p]], rows[p],
                                  semr[p]).wait()

            @plsc.parallel_loop(0, CH, unroll=2)
            def _(k):
                ex = bex[p][k, :]
                for g in range(4):
                    w = plsc.bitcast(rows[p][k, pl.ds(g * 16, 16)],
                                     jnp.int32)
                    lo = plsc.bitcast(w << 16, f32)
                    hi = plsc.bitcast(w & jnp.int32(-65536), f32)
                    frows[k, pl.ds(g * 16, 16)] = (
                        lo * jnp.full((16,), ex[g], f32))
                    frows[k, pl.ds(64 + g * 16, 16)] = (
                        hi * jnp.full((16,), ex[4 + g], f32))

            pltpu.sync_copy(frows, acc_sh.at[didxb[p]], add=True)
            issue_idx(i + 2, p)

    issue_idx(0, 0)
    issue_gathers(0, 0)
    issue_idx(1, 1)

    @pl.loop(0, (TPW + 1) // 2)
    def _(u):
        step(2 * u, 0)
        step(2 * u + 1, 1)

    plsc.subcore_barrier()

    @pl.when(core == 0)
    def _():
        dcopies = []
        for j in range(6):
            ro = rbase + j * CH
            dcopies.append(pltpu.async_copy(
                acc_sh.at[pl.ds(ro, CH)], acc0_hbm.at[pl.ds(ro, CH)], semz))
            dcopies.append(pltpu.async_copy(
                den_sh.at[pl.ds(ro, CH)], den0_hbm.at[pl.ds(ro, CH)], semz))
        ro2 = rbase + 6 * CH
        dcopies.append(pltpu.async_copy(
            acc_sh.at[pl.ds(ro2, PART)], acc0_hbm.at[pl.ds(ro2, PART)], semz))
        dcopies.append(pltpu.async_copy(
            den_sh.at[pl.ds(ro2, PART)], den0_hbm.at[pl.ds(ro2, PART)], semz))
        for cp in dcopies:
            cp.wait()

    @pl.when(core == 1)
    def _():
        dcopies = []
        for j in range(6):
            ro = rbase + j * CH
            dcopies.append(pltpu.async_copy(
                acc_sh.at[pl.ds(ro, CH)], acc1_hbm.at[pl.ds(ro, CH)], semz))
            dcopies.append(pltpu.async_copy(
                den_sh.at[pl.ds(ro, CH)], den1_hbm.at[pl.ds(ro, CH)], semz))
        ro2 = rbase + 6 * CH
        dcopies.append(pltpu.async_copy(
            acc_sh.at[pl.ds(ro2, PART)], acc1_hbm.at[pl.ds(ro2, PART)], semz))
        dcopies.append(pltpu.async_copy(
            den_sh.at[pl.ds(ro2, PART)], den1_hbm.at[pl.ds(ro2, PART)], semz))
        for cp in dcopies:
            cp.wait()


def _edge_kernel(xh, tsrc, tdst, ae_l, srcp, dstp, gv):
    mesh = plsc.VectorSubcoreMesh(core_axis_name="c", subcore_axis_name="s")
    kfn = pl.kernel(
        _edge_body,
        out_type=[
            jax.ShapeDtypeStruct((NP, HID), f32),
            jax.ShapeDtypeStruct((NP, HID), f32),
            jax.ShapeDtypeStruct((NP, 16), f32),
            jax.ShapeDtypeStruct((NP, 16), f32),
        ],
        mesh=mesh,
        scratch_types=[
            pltpu.VMEM_SHARED((NP, HID), f32),
            pltpu.VMEM_SHARED((NP, 16), f32),
            pltpu.VMEM((16,), f32),
            pltpu.VMEM((CH,), jnp.int32),
            pltpu.VMEM((CH,), jnp.int32),
            pltpu.VMEM((CH,), jnp.int32),
            pltpu.VMEM((CH,), jnp.int32),
            pltpu.VMEM((CH, 16), f32),
            pltpu.VMEM((CH, 16), f32),
            pltpu.VMEM((CH, 16), f32),
            pltpu.VMEM((CH, 16), f32),
            pltpu.VMEM((CH * 16,), f32),
            pltpu.VMEM((CH * 16,), f32),
            pltpu.VMEM((CH, 16), f32),
            pltpu.VMEM((CH, 16), f32),
            pltpu.VMEM((CH, 64), f32),
            pltpu.VMEM((CH, 64), f32),
            pltpu.VMEM((CH, HID), f32),
        ] + [pltpu.SemaphoreType.DMA] * 11,
        compiler_params=pltpu.CompilerParams(use_tc_tiling_on_sc=False,
                                             needs_layout_passes=False),
    )
    return kfn(xh, tsrc, tdst, ae_l, srcp, dstp, gv)


def kernel(x, edge_index, edge_attr, W_in, b_in, ln_in_g, ln_in_b, W_gat,
           b_gat, att_src, att_dst, W_edge, att_edge, ln_g, ln_b, W_h1, b_h1,
           W_h2, b_h2):
    eye8 = jnp.eye(H, dtype=f32)
    asf = (att_src[:, :, :, None] * eye8[:, None, :]).reshape(LAYERS, HID, H)
    adf = (att_dst[:, :, :, None] * eye8[:, None, :]).reshape(LAYERS, HID, H)
    acat = jnp.concatenate([asf, asf, adf, adf], axis=-1)
    me = jnp.einsum('ldhc,lhc->ldh',
                    W_edge.reshape(LAYERS, EDGE_DIM, H, C), att_edge)
    medup = jnp.concatenate([me, me], axis=-1)
    klist = [jnp.kron(eye8, medup[l]) for l in range(LAYERS)]
    eexp = jnp.concatenate(
        [jnp.repeat(eye8, 16, axis=1), jnp.zeros((8, HID), f32)], axis=0)
    w2p = jnp.zeros((HID, HID), f32).at[:, :OUT_DIM].set(W_h2)
    b2p = jnp.zeros((1, HID), f32).at[0, :OUT_DIM].set(b_h2)

    xp = jnp.concatenate([x, jnp.zeros((NP - N, HID), f32)], axis=0)
    fill = jnp.full((EP - E,), NP - 1, edge_index.dtype)
    srcp = jnp.concatenate([edge_index[0], fill])
    dstp = jnp.concatenate([edge_index[1], fill])
    eap = jnp.concatenate([edge_attr, jnp.zeros((EP - E, EDGE_DIM), f32)],
                          axis=0)

    ea_r = eap.reshape(EP // 8, 8 * EDGE_DIM)
    ae_out = _ae_kernel(ea_r, klist)
    ae_list = ae_out[:LAYERS]
    gmax_e = [jnp.max(ae_out[LAYERS + l], axis=(0, 1)).reshape(8, 16)
              .max(axis=0) for l in range(LAYERS)]

    h, xh, tsrc, tdst = _prologue(xp, W_in, b_in.reshape(1, HID),
                                  ln_in_g.reshape(1, HID),
                                  ln_in_b.reshape(1, HID), W_gat[0], acat[0])

    for l in range(LAYERS):
        gv = jnp.max(tsrc[:N], axis=0) + gmax_e[l]
        ae_flat = ae_list[l].reshape(EP * 16)
        a0, a1, d0, d1 = _edge_kernel(xh, tsrc, tdst, ae_flat, srcp, dstp, gv)
        bg = b_gat[l].reshape(1, HID)
        lg = ln_g[l].reshape(1, HID)
        lb = ln_b[l].reshape(1, HID)
        if l < LAYERS - 1:
            h, xh, tsrc, tdst = _postpre(h, a0, a1, d0, d1, eexp, bg, lg, lb,
                                         W_gat[l + 1], acat[l + 1])
        else:
            out = _postepi(h, a0, a1, d0, d1, eexp, bg, lg, lb,
                           W_h1, b_h1.reshape(1, HID), w2p, b2p)

    return out[:N, :OUT_DIM]
